# baseline (speedup 1.0000x reference)
import functools

import jax
import jax.numpy as jnp
from jax import lax
from jax.experimental import pallas as pl
from jax.experimental.pallas import tpu as pltpu
from jax.experimental.pallas import tpu_sc as plsc

D_MODEL = 2048
N_ROWS = 4 * 8192
_L = 16

_info = plsc.get_sparse_core_info()
_NC = _info.num_cores
_NS = _info.num_subcores
_NW = _NC * _NS
_BPW = N_ROWS // _NW
_C = 24
_NFULL = _BPW // _C
_TAIL = _BPW - _NFULL * _C

_mesh = plsc.VectorSubcoreMesh(core_axis_name="c", subcore_axis_name="s")


@functools.partial(
    pl.kernel,
    out_type=jax.ShapeDtypeStruct((N_ROWS, D_MODEL), jnp.float32),
    mesh=_mesh,
    scratch_types=[
        pltpu.VMEM((D_MODEL,), jnp.float32),
        pltpu.VMEM((D_MODEL,), jnp.float32),
        pltpu.VMEM((_BPW,), jnp.int32),
        pltpu.VMEM((_C // 8, 128), jnp.int32),
        pltpu.VMEM((_C, D_MODEL), jnp.float32),
        pltpu.VMEM((_C, D_MODEL), jnp.float32),
        pltpu.SemaphoreType.DMA,
        pltpu.SemaphoreType.DMA,
        pltpu.SemaphoreType.DMA,
    ],
)
def _embed_sc(x_hbm, tab_hbm, out_hbm, row0_v, row1_v, idx_v, xb_v, buf0,
              buf1, sem0, sem1, sem2):
    wid = lax.axis_index("s") * _NC + lax.axis_index("c")
    base = wid * _BPW
    pltpu.sync_copy(tab_hbm.at[0], row0_v)
    pltpu.sync_copy(tab_hbm.at[1], row1_v)
    pltpu.sync_copy(x_hbm.at[pl.ds(base, _BPW)], idx_v)

    def build(cb, buf, rows):
        preds = []
        for i0 in range(0, rows, _L):
            idxv = idx_v[pl.ds(cb + i0, _L)]
            zerov = idxv * 0
            for i in range(_L):
                r = i0 + i
                xb_v[r // 8, pl.ds((r % 8) * _L, _L)] = zerov + idxv[i]
        preds = [
            xb_v[i // 8, pl.ds((i % 8) * _L, _L)] != 0
            for i in range(rows)
        ]

        @plsc.parallel_loop(0, D_MODEL // _L, unroll=4)
        def dbody(d):
            off = pl.multiple_of(d * _L, _L)
            r0 = row0_v[pl.ds(off, _L)]
            r1 = row1_v[pl.ds(off, _L)]
            for i in range(rows):
                buf[i, pl.ds(off, _L)] = jnp.where(preds[i], r1, r0)

    def fire(cb, buf, sem, rows):
        gbase = pl.multiple_of(base + cb, 8)
        pltpu.async_copy(buf.at[pl.ds(0, rows)],
                         out_hbm.at[pl.ds(gbase, rows)], sem)

    def drain(buf, sem, rows):
        pltpu.make_async_copy(buf.at[pl.ds(0, rows)],
                              out_hbm.at[pl.ds(base, rows)], sem).wait()

    build(0, buf0, _C)
    fire(0, buf0, sem0, _C)
    build(_C, buf1, _C)
    fire(_C, buf1, sem1, _C)

    def pair(j, carry):
        cb = pl.multiple_of(j * (2 * _C), 8)
        drain(buf0, sem0, _C)
        build(cb, buf0, _C)
        fire(cb, buf0, sem0, _C)
        drain(buf1, sem1, _C)
        build(cb + _C, buf1, _C)
        fire(cb + _C, buf1, sem1, _C)
        return carry

    lax.fori_loop(1, _NFULL // 2, pair, 0)
    drain(buf0, sem0, _C)
    build(_NFULL * _C, buf0, _TAIL)
    fire(_NFULL * _C, buf0, sem2, _TAIL)
    drain(buf1, sem1, _C)
    drain(buf0, sem2, _TAIL)


def kernel(x, table):
    xf = x.reshape(-1).astype(jnp.int32)
    out = _embed_sc(xf, table)
    return out.reshape(x.shape[0], x.shape[1], D_MODEL)


# speedup vs baseline: 1.7664x; 1.7664x over previous
import functools

import jax
import jax.numpy as jnp
from jax import lax
from jax.experimental import pallas as pl
from jax.experimental.pallas import tpu as pltpu
from jax.experimental.pallas import tpu_sc as plsc

D_MODEL = 2048
N_ROWS = 4 * 8192
_L = 16

_info = plsc.get_sparse_core_info()
_NC = _info.num_cores
_NS = _info.num_subcores
_NW = _NC * _NS
_BPW = N_ROWS // _NW
_C = 16
_NPAIR = _BPW // (2 * _C)

_mesh = plsc.VectorSubcoreMesh(core_axis_name="c", subcore_axis_name="s")


@functools.partial(
    pl.kernel,
    out_type=jax.ShapeDtypeStruct((N_ROWS, D_MODEL), jnp.float32),
    mesh=_mesh,
    scratch_types=[
        pltpu.VMEM((D_MODEL,), jnp.float32),
        pltpu.VMEM((D_MODEL,), jnp.float32),
        pltpu.VMEM((_BPW,), jnp.int32),
        pltpu.VMEM((2, 128), jnp.int32),
        pltpu.VMEM((_C, D_MODEL), jnp.float32),
        pltpu.VMEM((_C, D_MODEL), jnp.float32),
        pltpu.SemaphoreType.DMA,
        pltpu.SemaphoreType.DMA,
    ],
)
def _embed_sc(x_hbm, tab_hbm, out_hbm, row0_v, row1_v, idx_v, xb_v, buf0,
              buf1, sem0, sem1):
    wid = lax.axis_index("s") * _NC + lax.axis_index("c")
    base = wid * _BPW
    pltpu.sync_copy(tab_hbm.at[0], row0_v)
    pltpu.sync_copy(tab_hbm.at[1], row1_v)
    pltpu.sync_copy(x_hbm.at[pl.ds(base, _BPW)], idx_v)

    def build(cb, buf):
        idxv = idx_v[pl.ds(cb, _L)]
        zerov = idxv * 0
        for i in range(_C):
            xb_v[i // 8, pl.ds((i % 8) * _L, _L)] = zerov + idxv[i]
        preds = [
            xb_v[i // 8, pl.ds((i % 8) * _L, _L)] != 0
            for i in range(_C)
        ]

        @plsc.parallel_loop(0, D_MODEL // _L, unroll=4)
        def dbody(d):
            off = pl.multiple_of(d * _L, _L)
            r0 = row0_v[pl.ds(off, _L)]
            r1 = row1_v[pl.ds(off, _L)]
            for i in range(_C):
                buf[i, pl.ds(off, _L)] = jnp.where(preds[i], r1, r0)

    def fire(cb, buf, sem):
        gbase = pl.multiple_of(base + cb, 8)
        pltpu.async_copy(buf, out_hbm.at[pl.ds(gbase, _C)], sem)

    def drain(buf, sem):
        pltpu.make_async_copy(buf, out_hbm.at[pl.ds(base, _C)], sem).wait()

    build(0, buf0)
    fire(0, buf0, sem0)
    build(_C, buf1)
    fire(_C, buf1, sem1)

    def pair(j, carry):
        cb = pl.multiple_of(j * (2 * _C), _C)
        drain(buf0, sem0)
        build(cb, buf0)
        fire(cb, buf0, sem0)
        drain(buf1, sem1)
        build(cb + _C, buf1)
        fire(cb + _C, buf1, sem1)
        return carry

    lax.fori_loop(1, _NPAIR, pair, 0)
    drain(buf0, sem0)
    drain(buf1, sem1)


def kernel(x, table):
    xf = x.reshape(-1).astype(jnp.int32)
    out = _embed_sc(xf, table)
    return out.reshape(x.shape[0], x.shape[1], D_MODEL)
